# baseline (device time: 105158 ns/iter reference)
import jax
import jax.numpy as jnp
from jax import lax
from jax.experimental import pallas as pl
from jax.experimental.pallas import tpu as pltpu

N_DEV = 16
MESH = pl.DeviceIdType.MESH
H = 4
DEPTH = 3

R_LAST = 15
L_LAST = 13


def _gelu(y):
    c = 0.7978845608028654
    return 0.5 * y * (1.0 + jnp.tanh(c * (y + 0.044715 * y * y * y)))


def kernel(x, w_mat):
    m, k_per = x.shape
    _, n = w_mat.shape
    chunk = m // N_DEV
    cw = n // H
    bf16 = jnp.bfloat16
    f32 = jnp.float32

    def body(x_ref, w_ref, out_ref, *scratch):
        comms = scratch[0:2 * H]
        ssems = scratch[2 * H:4 * H]
        rsems = scratch[4 * H:6 * H]
        creds = scratch[6 * H:8 * H]

        my = lax.axis_index("i")

        p = lax.rem(my, 4)
        z = lax.div(my, 4)
        q = jnp.where(
            p == 0, z,
            jnp.where(p == 3, 7 - z, jnp.where(p == 2, 8 + z, 15 - z)),
        )

        def perm(r):
            return jnp.where(
                r < 4, 4 * r,
                jnp.where(r < 8, 31 - 4 * r,
                          jnp.where(r < 12, 4 * r - 30, 61 - 4 * r)),
            )

        left = perm(lax.rem(q + N_DEV - 1, N_DEV))
        right = perm(lax.rem(q + 1, N_DEV))

        def row(off):
            return lax.rem(q + off + 2 * N_DEV, N_DEV) * chunk

        def matmul(off):
            r0 = row(off)
            out_ref[pl.ds(r0, chunk), :] = jnp.dot(
                x_ref[pl.ds(r0, chunk), :], w_ref[...],
                preferred_element_type=f32,
            )

        def acc_q(off, co):
            return out_ref[pl.ds(row(off), chunk), co:co + cw]

        class Lane:
            def __init__(self, i, dst, cred_to, co, last):
                self.comm, self.ssem = comms[i], ssems[i]
                self.rsem, self.cred = rsems[i], creds[i]
                self.dst, self.cred_to, self.co, self.last = dst, cred_to, co, last

            def mk(self, k):
                return pltpu.make_async_remote_copy(
                    src_ref=self.comm.at[k % DEPTH],
                    dst_ref=self.comm.at[(k + 1) % DEPTH],
                    send_sem=self.ssem.at[k % DEPTH],
                    recv_sem=self.rsem.at[(k + 1) % DEPTH],
                    device_id=(self.dst,),
                    device_id_type=MESH,
                )

            def issue(self, k):
                if k >= DEPTH - 1:
                    pl.semaphore_wait(self.cred, 1)
                self.mk(k).start()

            def finish_send(self, k):
                self.mk(k).wait_send()
                if k <= self.last - (DEPTH - 1):
                    pl.semaphore_signal(self.cred, inc=1,
                                        device_id=(self.cred_to,),
                                        device_id_type=MESH)

        r_lanes = [Lane(j, right, left, j * cw, R_LAST) for j in range(H)]
        l_lanes = [Lane(H + j, left, right, j * cw, L_LAST) for j in range(H)]

        def add_r(lane, k):
            pass

        def add_l(lane, k):
            pass

        def store_r(lane, k):
            pass

        def store_l(lane, k):
            pass

        def combine(r_lane, l_lane):
            r_fin = (7 + 1) % DEPTH
            l_fin = (6 + 1) % DEPTH
            co = r_lane.co
            pass

        matmul(8)
        matmul(-7)

        barrier = pltpu.get_barrier_semaphore()
        for nbr in (left, right):
            pl.semaphore_signal(barrier, inc=1, device_id=(nbr,),
                                device_id_type=MESH)
        pl.semaphore_wait(barrier, 2)

        for lane in r_lanes:
            lane.comm[0, :, :] = acc_q(8, lane.co).astype(bf16)
        for lane in l_lanes:
            lane.comm[0, :, :] = acc_q(-7, lane.co).astype(bf16)

        pend_r = pend_l = None
        for s in range(17):
            ra, la = r_lanes[0], l_lanes[0]
            if s <= R_LAST:
                ra.issue(s)
            if 1 <= s <= L_LAST + 1:
                la.issue(s - 1)
            if s == 0:
                for off in (7, -6, 6, -5, 5, -4, 4, -3, 3, -2, 2, -1, 1, 0):
                    matmul(off)
            if pend_r is not None:
                store_r(ra, pend_r)
                pend_r = None
            if pend_l is not None:
                store_l(la, pend_l)
                pend_l = None
            for j in range(1, H):
                rj, lj = r_lanes[j], l_lanes[j]
                if s == 8:
                    rj.finish_send(7)
                    rj.mk(7).wait_recv()
                    lj.finish_send(6)
                    lj.mk(6).wait_recv()
                    combine(rj, lj)
                    rj.issue(8)
                    lj.issue(7)
                else:
                    if 1 <= s <= R_LAST + 1:
                        rj.finish_send(s - 1)
                        rj.mk(s - 1).wait_recv()
                        if s - 1 <= 6:
                            add_r(rj, s - 1)
                    if s <= R_LAST:
                        rj.issue(s)
                    if 2 <= s <= L_LAST + 2:
                        lj.finish_send(s - 2)
                        lj.mk(s - 2).wait_recv()
                        if s - 2 <= 5:
                            add_l(lj, s - 2)
                    if 1 <= s <= L_LAST + 1:
                        lj.issue(s - 1)
                if 1 <= s <= R_LAST + 1 and s - 1 >= 8:
                    store_r(rj, s - 1)
                if 2 <= s <= L_LAST + 2 and s - 2 >= 7:
                    store_l(lj, s - 2)
            if s <= R_LAST:
                ra.finish_send(s)
                ra.mk(s).wait_recv()
                if s <= 6:
                    add_r(ra, s)
                elif s >= 8:
                    pend_r = s
            if 1 <= s <= L_LAST + 1:
                la.finish_send(s - 1)
                la.mk(s - 1).wait_recv()
                if s - 1 <= 5:
                    add_l(la, s - 1)
                elif s - 1 >= 7:
                    pend_l = s - 1
            if s == 7:
                combine(ra, la)

    comm_shape = pltpu.VMEM((DEPTH, chunk, cw), bf16)
    dma = pltpu.SemaphoreType.DMA((DEPTH,))
    reg = pltpu.SemaphoreType.REGULAR
    return pl.pallas_call(
        body,
        out_shape=jax.ShapeDtypeStruct((m, n), f32),
        in_specs=[
            pl.BlockSpec(memory_space=pltpu.VMEM),
            pl.BlockSpec(memory_space=pltpu.VMEM),
        ],
        out_specs=pl.BlockSpec(memory_space=pltpu.VMEM),
        scratch_shapes=(
            [comm_shape] * (2 * H)
            + [dma] * (2 * H)
            + [dma] * (2 * H)
            + [reg] * (2 * H)
        ),
        compiler_params=pltpu.CompilerParams(collective_id=0),
    )(x, w_mat)


# device time: 100802 ns/iter; 1.0432x vs baseline; 1.0432x over previous
import jax
import jax.numpy as jnp
from jax import lax
from jax.experimental import pallas as pl
from jax.experimental.pallas import tpu as pltpu

N_DEV = 16
MESH = pl.DeviceIdType.MESH
H = 4
DEPTH = 3

LONG_LAST = 15
SHORT_LAST = 13


def _gelu(y):
    c = 0.7978845608028654
    return 0.5 * y * (1.0 + jnp.tanh(c * (y + 0.044715 * y * y * y)))


def kernel(x, w_mat):
    m, k_per = x.shape
    _, n = w_mat.shape
    chunk = m // N_DEV
    cw = n // H
    bf16 = jnp.bfloat16
    f32 = jnp.float32

    def body(x_ref, w_ref, out_ref, *scratch):
        comms = scratch[0:2 * H]
        ssems = scratch[2 * H:4 * H]
        rsems = scratch[4 * H:6 * H]
        creds = scratch[6 * H:8 * H]

        my = lax.axis_index("i")

        p = lax.rem(my, 4)
        z = lax.div(my, 4)
        q = jnp.where(
            p == 0, z,
            jnp.where(p == 3, 7 - z, jnp.where(p == 2, 8 + z, 15 - z)),
        )

        def perm(r):
            return jnp.where(
                r < 4, 4 * r,
                jnp.where(r < 8, 31 - 4 * r,
                          jnp.where(r < 12, 4 * r - 30, 61 - 4 * r)),
            )

        left = perm(lax.rem(q + N_DEV - 1, N_DEV))
        right = perm(lax.rem(q + 1, N_DEV))

        def row(off):
            return lax.rem(q + off + 2 * N_DEV, N_DEV) * chunk

        def matmul(off):
            r0 = row(off)
            out_ref[pl.ds(r0, chunk), :] = jnp.dot(
                x_ref[pl.ds(r0, chunk), :], w_ref[...],
                preferred_element_type=f32,
            )

        def acc_q(off, co):
            return out_ref[pl.ds(row(off), chunk), co:co + cw]

        class Lane:
            def __init__(self, i, dst, cred_to, sign, co, last):
                self.comm, self.ssem = comms[i], ssems[i]
                self.rsem, self.cred = rsems[i], creds[i]
                self.dst, self.cred_to = dst, cred_to
                self.sign, self.co, self.last = sign, co, last

            def mk(self, k):
                return pltpu.make_async_remote_copy(
                    src_ref=self.comm.at[k % DEPTH],
                    dst_ref=self.comm.at[(k + 1) % DEPTH],
                    send_sem=self.ssem.at[k % DEPTH],
                    recv_sem=self.rsem.at[(k + 1) % DEPTH],
                    device_id=(self.dst,),
                    device_id_type=MESH,
                )

            def issue(self, k):
                if k >= DEPTH - 1:
                    pl.semaphore_wait(self.cred, 1)
                self.mk(k).start()

            def finish_send(self, k):
                self.mk(k).wait_send()
                if k <= self.last - (DEPTH - 1):
                    pl.semaphore_signal(self.cred, inc=1,
                                        device_id=(self.cred_to,),
                                        device_id_type=MESH)

            def far(self):
                return 8 if self.last == LONG_LAST else 7

            def seed(self):
                self.comm[0, :, :] = (
                    acc_q(self.far() * self.sign, self.co).astype(bf16)
                )

            def add(self, k):
                rs = (k + 1) % DEPTH
                off = (self.far() - 1 - k) * self.sign
                self.comm[rs, :, :] = (
                    self.comm[rs, :, :] + acc_q(off, self.co).astype(bf16)
                )

            def store(self, k):
                rs = (k + 1) % DEPTH
                off = (self.far() - 1 - k) * self.sign
                out_ref[pl.ds(row(off), chunk), self.co:self.co + cw] = (
                    self.comm[rs, :, :].astype(f32)
                )

        r_long_lead = Lane(0, right, left, 1, 0 * cw, LONG_LAST)
        r_long_trail = Lane(1, right, left, 1, 1 * cw, LONG_LAST)
        r_short_lead = Lane(2, right, left, 1, 2 * cw, SHORT_LAST)
        r_short_trail = Lane(3, right, left, 1, 3 * cw, SHORT_LAST)
        l_short_lead = Lane(4, left, right, -1, 0 * cw, SHORT_LAST)
        l_short_trail = Lane(5, left, right, -1, 1 * cw, SHORT_LAST)
        l_long_lead = Lane(6, left, right, -1, 2 * cw, LONG_LAST)
        l_long_trail = Lane(7, left, right, -1, 3 * cw, LONG_LAST)

        long_leads = [r_long_lead, l_long_lead]
        long_trails = [r_long_trail, l_long_trail]
        short_leads = [l_short_lead, r_short_lead]
        short_trails = [l_short_trail, r_short_trail]
        lead_pairs = [(r_long_lead, l_short_lead),
                      (l_long_lead, r_short_lead)]
        trail_pairs = [(r_long_trail, l_short_trail),
                       (l_long_trail, r_short_trail)]

        def combine(long_lane, short_lane):
            l_fin = (7 + 1) % DEPTH
            s_fin = (6 + 1) % DEPTH
            co = long_lane.co
            total = (long_lane.comm[l_fin, :, :].astype(f32)
                     + short_lane.comm[s_fin, :, :].astype(f32)
                     + acc_q(0, co))
            g = _gelu(total)
            out_ref[pl.ds(row(0), chunk), co:co + cw] = g
            gb = g.astype(bf16)
            long_lane.comm[l_fin, :, :] = gb
            short_lane.comm[s_fin, :, :] = gb

        matmul(8)
        matmul(7)
        matmul(-7)

        barrier = pltpu.get_barrier_semaphore()
        for nbr in (left, right):
            pl.semaphore_signal(barrier, inc=1, device_id=(nbr,),
                                device_id_type=MESH)
        pl.semaphore_wait(barrier, 2)

        for lane in long_leads + long_trails + short_leads + short_trails:
            lane.seed()

        pends = {}
        for s in range(17):
            for lane in long_leads:
                if s <= LONG_LAST:
                    lane.issue(s)
            for lane in short_leads:
                if 1 <= s <= SHORT_LAST + 1:
                    lane.issue(s - 1)
            if s == 0:
                for off in (6, -6, 5, -5, 4, -4, 3, -3, 2, -2, 1, -1, 0):
                    matmul(off)
            for lane, k in pends.items():
                lane.store(k)
            pends = {}
            for lane in long_trails:
                if 1 <= s <= LONG_LAST + 1:
                    lane.finish_send(s - 1)
                    lane.mk(s - 1).wait_recv()
                    if s - 1 <= 6:
                        lane.add(s - 1)
            for lane in short_trails:
                if 2 <= s <= SHORT_LAST + 2:
                    lane.finish_send(s - 2)
                    lane.mk(s - 2).wait_recv()
                    if s - 2 <= 5:
                        lane.add(s - 2)
            if s == 8:
                for lg, sh in trail_pairs:
                    combine(lg, sh)
            for lane in long_trails:
                if s <= LONG_LAST:
                    lane.issue(s)
            for lane in short_trails:
                if 1 <= s <= SHORT_LAST + 1:
                    lane.issue(s - 1)
            for lane in long_trails:
                if 1 <= s <= LONG_LAST + 1 and s - 1 >= 8:
                    lane.store(s - 1)
            for lane in short_trails:
                if 2 <= s <= SHORT_LAST + 2 and s - 2 >= 7:
                    lane.store(s - 2)
            for lane in long_leads:
                if s <= LONG_LAST:
                    lane.finish_send(s)
                    lane.mk(s).wait_recv()
                    if s <= 6:
                        lane.add(s)
                    elif s >= 8:
                        pends[lane] = s
            for lane in short_leads:
                if 1 <= s <= SHORT_LAST + 1:
                    lane.finish_send(s - 1)
                    lane.mk(s - 1).wait_recv()
                    if s - 1 <= 5:
                        lane.add(s - 1)
                    elif s - 1 >= 7:
                        pends[lane] = s - 1
            if s == 7:
                for lg, sh in lead_pairs:
                    combine(lg, sh)

    comm_shape = pltpu.VMEM((DEPTH, chunk, cw), bf16)
    dma = pltpu.SemaphoreType.DMA((DEPTH,))
    reg = pltpu.SemaphoreType.REGULAR
    return pl.pallas_call(
        body,
        out_shape=jax.ShapeDtypeStruct((m, n), f32),
        in_specs=[
            pl.BlockSpec(memory_space=pltpu.VMEM),
            pl.BlockSpec(memory_space=pltpu.VMEM),
        ],
        out_specs=pl.BlockSpec(memory_space=pltpu.VMEM),
        scratch_shapes=(
            [comm_shape] * (2 * H)
            + [dma] * (2 * H)
            + [dma] * (2 * H)
            + [reg] * (2 * H)
        ),
        compiler_params=pltpu.CompilerParams(collective_id=0),
    )(x, w_mat)
